# baseline (speedup 1.0000x reference)
import math

import jax
import jax.numpy as jnp
from jax.experimental import pallas as pl
from jax.experimental.pallas import tpu as pltpu
from jax.experimental.pallas import tpu_sc as plsc

_LANES = 16


def _sc_pe_add(x2d, pe, seq, scale):
    rows, d = x2d.shape
    bs = 16
    n_seq_blocks = seq // bs
    grid = (rows // bs,)

    mesh = plsc.VectorSubcoreMesh(core_axis_name="core", subcore_axis_name="subcore")

    @pl.kernel(out_type=jax.ShapeDtypeStruct((rows, d), x2d.dtype), mesh=mesh)
    def sc_kernel(x_hbm, pe_hbm, o_hbm):
        def body(x_vmem, pe_vmem, o_vmem):
            @pl.loop(0, bs)
            def _row(r):
                @pl.loop(0, d, step=_LANES)
                def _col(c):
                    slc = (pl.ds(r, 1), pl.ds(c, _LANES))
                    o_vmem.at[*slc][...] = (
                        x_vmem.at[*slc][...] * scale + pe_vmem.at[*slc][...]
                    )

        pltpu.emit_pipeline(
            body,
            grid=grid,
            in_specs=[
                pl.BlockSpec((bs, d), index_map=lambda i: (i, 0)),
                pl.BlockSpec((bs, d), index_map=lambda i: (i % n_seq_blocks, 0)),
            ],
            out_specs=[pl.BlockSpec((bs, d), index_map=lambda i: (i, 0))],
            core_axis_name=("core", "subcore"),
            dimension_semantics=(pltpu.PARALLEL,),
        )(x_hbm, pe_hbm, o_hbm)

    return sc_kernel(x2d, pe)


def kernel(input, pe):
    batch, seq, d_model = input.shape
    scale = math.sqrt(pe.shape[1])
    x2d = input.reshape(batch * seq, d_model)
    out = _sc_pe_add(x2d, pe, seq, scale)
    return out.reshape(batch, seq, d_model)


# speedup vs baseline: 1.8653x; 1.8653x over previous
import math

import jax
import jax.numpy as jnp
from jax.experimental import pallas as pl
from jax.experimental.pallas import tpu as pltpu
from jax.experimental.pallas import tpu_sc as plsc

_LANES = 16
_S0 = 3072


def _tc_pe_add_kernel(x_ref, pe_ref, o_ref, *, scale):
    o_ref[...] = x_ref[...] * scale + pe_ref[...][None, :, :]


def _tc_part(input, pe, scale):
    batch, seq, d = input.shape
    blk = 512
    return pl.pallas_call(
        lambda x_ref, pe_ref, o_ref: _tc_pe_add_kernel(x_ref, pe_ref, o_ref, scale=scale),
        grid=(_S0 // blk,),
        in_specs=[
            pl.BlockSpec((batch, blk, d), lambda i: (0, i, 0)),
            pl.BlockSpec((blk, d), lambda i: (i, 0)),
        ],
        out_specs=pl.BlockSpec((batch, blk, d), lambda i: (0, i, 0)),
        out_shape=jax.ShapeDtypeStruct((batch, _S0, d), input.dtype),
    )(input, pe)


def _sc_part(input, pe, scale):
    batch, seq, d = input.shape
    bs = 4
    n_blocks = (seq - _S0) // bs
    mesh = plsc.VectorSubcoreMesh(core_axis_name="core", subcore_axis_name="subcore")

    @pl.kernel(
        out_type=jax.ShapeDtypeStruct((batch, seq - _S0, d), input.dtype),
        mesh=mesh,
    )
    def sc_kernel(x_hbm, pe_hbm, o_hbm):
        def body(x_vmem, pe_vmem, o_vmem):
            @pl.loop(0, bs)
            def _row(r):
                @pl.loop(0, d, step=_LANES)
                def _col(c):
                    slc = (pl.ds(r, 1), pl.ds(c, _LANES))
                    pv = pe_vmem.at[*slc][...]
                    for b in range(batch):
                        o_vmem.at[b].at[*slc][...] = (
                            x_vmem.at[b].at[*slc][...] * scale + pv
                        )

        pltpu.emit_pipeline(
            body,
            grid=(n_blocks,),
            in_specs=[
                pl.BlockSpec((batch, bs, d), index_map=lambda i: (0, _S0 // bs + i, 0)),
                pl.BlockSpec((bs, d), index_map=lambda i: (_S0 // bs + i, 0)),
            ],
            out_specs=[
                pl.BlockSpec((batch, bs, d), index_map=lambda i: (0, i, 0))
            ],
            core_axis_name=("core", "subcore"),
            dimension_semantics=(pltpu.PARALLEL,),
        )(x_hbm, pe_hbm, o_hbm)

    return sc_kernel(input, pe)


def kernel(input, pe):
    scale = math.sqrt(pe.shape[1])
    tc_out = _tc_part(input, pe, scale)
    sc_out = _sc_part(input, pe, scale)
    return jnp.concatenate([tc_out, sc_out], axis=1)


# speedup vs baseline: 4.1785x; 2.2401x over previous
import math

import jax
import jax.numpy as jnp
from jax.experimental import pallas as pl


def _pe_add_kernel(x_ref, pe_ref, o_ref, *, scale):
    o_ref[...] = x_ref[...] * scale + pe_ref[...][None, :, :]


def kernel(input, pe):
    batch, seq, d_model = input.shape
    scale = math.sqrt(pe.shape[1])
    blk = 512
    grid = (seq // blk, batch)
    return pl.pallas_call(
        lambda x_ref, pe_ref, o_ref: _pe_add_kernel(x_ref, pe_ref, o_ref, scale=scale),
        grid=grid,
        in_specs=[
            pl.BlockSpec((1, blk, d_model), lambda i, b: (b, i, 0)),
            pl.BlockSpec((blk, d_model), lambda i, b: (i, 0)),
        ],
        out_specs=pl.BlockSpec((1, blk, d_model), lambda i, b: (b, i, 0)),
        out_shape=jax.ShapeDtypeStruct((batch, seq, d_model), input.dtype),
    )(input, pe)


# speedup vs baseline: 4.7116x; 1.1276x over previous
import math

import jax
import jax.numpy as jnp
from jax.experimental import pallas as pl


def _pe_add_kernel(x_ref, pe_ref, o_ref, *, scale):
    o_ref[...] = x_ref[...] * scale + pe_ref[...][None, :, :]


def kernel(input, pe):
    batch, seq, d_model = input.shape
    scale = math.sqrt(pe.shape[1])
    blk = 256
    grid = (seq // blk,)
    return pl.pallas_call(
        lambda x_ref, pe_ref, o_ref: _pe_add_kernel(x_ref, pe_ref, o_ref, scale=scale),
        grid=grid,
        in_specs=[
            pl.BlockSpec((batch, blk, d_model), lambda i: (0, i, 0)),
            pl.BlockSpec((blk, d_model), lambda i: (i, 0)),
        ],
        out_specs=pl.BlockSpec((batch, blk, d_model), lambda i: (0, i, 0)),
        out_shape=jax.ShapeDtypeStruct((batch, seq, d_model), input.dtype),
    )(input, pe)


# speedup vs baseline: 4.7974x; 1.0182x over previous
import math

import jax
import jax.numpy as jnp
from jax.experimental import pallas as pl


def _pe_add_kernel(x_ref, pe_ref, o_ref, *, scale):
    o_ref[...] = x_ref[...] * scale + pe_ref[...][None, :, :]


def kernel(input, pe):
    batch, seq, d_model = input.shape
    scale = math.sqrt(pe.shape[1])
    blk = 512
    bd = d_model // 2
    grid = (seq // blk, 2)
    return pl.pallas_call(
        lambda x_ref, pe_ref, o_ref: _pe_add_kernel(x_ref, pe_ref, o_ref, scale=scale),
        grid=grid,
        in_specs=[
            pl.BlockSpec((batch, blk, bd), lambda i, j: (0, i, j)),
            pl.BlockSpec((blk, bd), lambda i, j: (i, j)),
        ],
        out_specs=pl.BlockSpec((batch, blk, bd), lambda i, j: (0, i, j)),
        out_shape=jax.ShapeDtypeStruct((batch, seq, d_model), input.dtype),
    )(input, pe)


# speedup vs baseline: 4.8439x; 1.0097x over previous
import math

import jax
import jax.numpy as jnp
from jax.experimental import pallas as pl


def _pe_add_kernel(x_ref, pe_ref, o_ref, *, scale):
    o_ref[...] = x_ref[...] * scale + pe_ref[...][None, :, :]


def kernel(input, pe):
    batch, seq, d_model = input.shape
    scale = math.sqrt(pe.shape[1])
    blk = 512
    grid = (seq // blk,)
    return pl.pallas_call(
        lambda x_ref, pe_ref, o_ref: _pe_add_kernel(x_ref, pe_ref, o_ref, scale=scale),
        grid=grid,
        in_specs=[
            pl.BlockSpec((batch, blk, d_model), lambda i: (0, i, 0)),
            pl.BlockSpec((blk, d_model), lambda i: (i, 0)),
        ],
        out_specs=pl.BlockSpec((batch, blk, d_model), lambda i: (0, i, 0)),
        out_shape=jax.ShapeDtypeStruct((batch, seq, d_model), input.dtype),
    )(input, pe)
